# baseline (speedup 1.0000x reference)
import functools

import jax
import jax.numpy as jnp
from jax import lax
from jax.experimental import pallas as pl
from jax.experimental.pallas import tpu as pltpu
from jax.experimental.pallas import tpu_sc as plsc

L = 16
NSLOTS = 4
DIST = 2


def _make_sc_kernel(B, H, W, C):
    mesh = plsc.VectorSubcoreMesh(core_axis_name="c", subcore_axis_name="s")
    n_vec = C // L

    @functools.partial(
        pl.kernel,
        mesh=mesh,
        out_type=jax.ShapeDtypeStruct((B, H, W, C), jnp.float32),
        scratch_types=[
            pltpu.VMEM((NSLOTS, W, C), jnp.float32),
            pltpu.VMEM((W, C), jnp.float32),
            pltpu.VMEM((C,), jnp.float32),
        ]
        + [pltpu.SemaphoreType.DMA] * (2 * NSLOTS),
    )
    def sc_kernel(x_hbm, row_hbm, col_hbm, out_hbm, buf, pos, rowv, *sems):
        isems = sems[:NSLOTS]
        osems = sems[NSLOTS:]
        h = lax.axis_index("s") * 2 + lax.axis_index("c")

        pltpu.sync_copy(col_hbm, pos)
        pltpu.sync_copy(row_hbm.at[h], rowv)

        @plsc.parallel_loop(0, W, unroll=2)
        def _pos_body(w):
            for j in range(n_vec):
                sl = pl.ds(j * L, L)
                plsc.addupdate(pos.at[w, sl], rowv[sl])

        for s in range(DIST):
            pltpu.async_copy(x_hbm.at[s, h], buf.at[s], isems[s])

        def group(g, carry):
            for s in (0, 2):
                b = NSLOTS * g + s
                s0, s1 = s, s + 1
                p0, p1 = (s + 2) % NSLOTS, (s + 3) % NSLOTS

                pltpu.make_async_copy(
                    x_hbm.at[b, h], buf.at[s0], isems[s0]).wait()
                pltpu.make_async_copy(
                    x_hbm.at[b + 1, h], buf.at[s1], isems[s1]).wait()

                def retire():
                    pltpu.make_async_copy(
                        buf.at[p0], out_hbm.at[b - 2, h], osems[p0]).wait()
                    pltpu.make_async_copy(
                        buf.at[p1], out_hbm.at[b - 1, h], osems[p1]).wait()

                if s == 0:
                    pl.when(g > 0)(retire)
                else:
                    retire()

                @pl.when(b + 2 < B)
                def _prefetch():
                    pltpu.async_copy(
                        x_hbm.at[b + 2, h], buf.at[p0], isems[p0])
                    pltpu.async_copy(
                        x_hbm.at[b + 3, h], buf.at[p1], isems[p1])

                @plsc.parallel_loop(0, W, unroll=2)
                def _add_body(w):
                    for j in range(n_vec):
                        sl = pl.ds(j * L, L)
                        pv = pos[w, sl]
                        plsc.addupdate(buf.at[s0, w, sl], pv)
                        plsc.addupdate(buf.at[s1, w, sl], pv)

                pltpu.async_copy(buf.at[s0], out_hbm.at[b, h], osems[s0])
                pltpu.async_copy(buf.at[s1], out_hbm.at[b + 1, h], osems[s1])
            return carry

        lax.fori_loop(0, B // NSLOTS, group, 0)

        for d in range(DIST, 0, -1):
            s = (B - d) % NSLOTS
            pltpu.make_async_copy(
                buf.at[s], out_hbm.at[B - d, h], osems[s]).wait()

    return sc_kernel


def kernel(x, row_emb, col_emb):
    b, h, w, c = x.shape
    return _make_sc_kernel(b, h, w, c)(x, row_emb, col_emb)
